# baseline (speedup 1.0000x reference)
import functools

import jax
import jax.numpy as jnp
from jax import lax
from jax.experimental import pallas as pl
from jax.experimental.pallas import tpu as pltpu
from jax.experimental.pallas import tpu_sc as plsc

H = 16
D = 64
K = 64
HALF_S = 16
S = 2 * HALF_S + 1
MAX_F = 16.0
MIN_F = 1.0
HALO = int(HALF_S * MAX_F + MAX_F)

NC = 2
NS = 16
NW = NC * NS

BL = 256
W = BL + 2 * HALO
ILP = 49

_RNE_MAGIC = 12582912.0


def _silu(v):
    return v * jax.nn.sigmoid(v)


def _proj_body(x_ref, wkT_ref, bk_ref, wwT_ref, bw_ref,
               kern_ref, freq_ref, phase_ref):
    xb = x_ref[...]
    kern_ref[...] = _silu(
        jnp.dot(xb, wkT_ref[...], preferred_element_type=jnp.float32)
        + bk_ref[...])
    wave = _silu(
        jnp.dot(xb, wwT_ref[...], preferred_element_type=jnp.float32)
        + bw_ref[...])
    freq_ref[...] = jax.nn.sigmoid(wave[:, :H]) * (MAX_F - MIN_F) + MIN_F
    phase_ref[...] = jnp.tanh(wave[:, H:]) * MAX_F


def _projections(x2d, WkT, bk, WwT, bw, BM):
    M, C = x2d.shape
    grid = (M // BM,)
    return pl.pallas_call(
        _proj_body,
        grid=grid,
        in_specs=[
            pl.BlockSpec((BM, C), lambda i: (i, 0)),
            pl.BlockSpec((C, H * K), lambda i: (0, 0)),
            pl.BlockSpec((1, H * K), lambda i: (0, 0)),
            pl.BlockSpec((C, 2 * H), lambda i: (0, 0)),
            pl.BlockSpec((1, 2 * H), lambda i: (0, 0)),
        ],
        out_specs=[
            pl.BlockSpec((BM, H * K), lambda i: (i, 0)),
            pl.BlockSpec((BM, H), lambda i: (i, 0)),
            pl.BlockSpec((BM, H), lambda i: (i, 0)),
        ],
        out_shape=[
            jax.ShapeDtypeStruct((M, H * K), jnp.float32),
            jax.ShapeDtypeStruct((M, H), jnp.float32),
            jax.ShapeDtypeStruct((M, H), jnp.float32),
        ],
    )(x2d, WkT, bk, WwT, bw)


def _out_body(h_ref, woT_ref, o_ref):
    o_ref[...] = _silu(
        jnp.dot(h_ref[...], woT_ref[...], preferred_element_type=jnp.float32))


def _out_proj(h2d, WoT, BM):
    M, C = h2d.shape
    return pl.pallas_call(
        _out_body,
        grid=(M // BM,),
        in_specs=[
            pl.BlockSpec((BM, C), lambda i: (i, 0)),
            pl.BlockSpec((C, C), lambda i: (0, 0)),
        ],
        out_specs=pl.BlockSpec((BM, C), lambda i: (i, 0)),
        out_shape=jax.ShapeDtypeStruct((M, C), jnp.float32),
    )(h2d, WoT)


def _gconv_body(L, nb, xt_ref, kw_ref, fq_ref, ph_ref, hid_ref,
                win, kwv, fqv, phv, outv, ilbuf):
    wid = lax.axis_index("s") * NC + lax.axis_index("c")
    iota = lax.iota(jnp.int32, 16)
    cols = [iota + 16 * j for j in range(4)]

    def block(blk, carry):
        l0 = blk * BL
        s0 = jnp.clip(l0 - HALO, 0, L - W)
        woff = pl.multiple_of(wid * (L * D) + s0 * D, 64)
        pltpu.sync_copy(xt_ref.at[pl.ds(woff, W * D)], win)
        koff = pl.multiple_of((wid * L + l0) * K, 256)
        pltpu.sync_copy(kw_ref.at[pl.ds(koff, BL * K)], kwv)
        boff = pl.multiple_of(wid * L + l0, 256)
        pltpu.sync_copy(fq_ref.at[pl.ds(boff, BL)], fqv)
        pltpu.sync_copy(ph_ref.at[pl.ds(boff, BL)], phv)

        def chunk(c, carry2):
            c16 = c * 16
            lvec = iota.astype(jnp.float32) + (l0 + c16).astype(jnp.float32)
            f = fqv[pl.ds(c16, 16)]
            p = phv[pl.ds(c16, 16)]
            base = lvec + p

            for s in range(S):
                pos = base + jnp.float32(s - HALF_S) * f
                r = (pos + _RNE_MAGIC) - _RNE_MAGIC
                r = jnp.clip(r, 0.0, jnp.float32(L - 1))
                il = r.astype(jnp.int32) - s0
                plsc.store_scatter(ilbuf, [iota * ILP + s], il * D)

            @plsc.parallel_loop(0, 16)
            def per_l(l):
                ilv = [ilbuf[pl.ds(l * ILP + 16 * t, 16)] for t in range(3)]
                wrow = [kwv[pl.ds((c16 + l) * K + 16 * t, 16)]
                        for t in range(3)]
                accs = [jnp.zeros((16,), jnp.float32) for _ in range(8)]
                for s in range(S):
                    ilb = jnp.full((16,), ilv[s // 16][s % 16], jnp.int32)
                    w = wrow[s // 16][s % 16]
                    for j in range(4):
                        v = plsc.load_gather(win, [ilb + cols[j]])
                        accs[(s % 2) * 4 + j] = accs[(s % 2) * 4 + j] + w * v
                for j in range(4):
                    outv[pl.ds((c16 + l) * D + 16 * j, 16)] = (
                        accs[j] + accs[4 + j])

            return carry2

        lax.fori_loop(0, BL // 16, chunk, 0)
        ooff = pl.multiple_of((wid * L + l0) * D, 256)
        pltpu.sync_copy(outv, hid_ref.at[pl.ds(ooff, BL * D)])
        return carry

    lax.fori_loop(0, nb, block, 0)


def _gather_conv(xt_flat, kwt, fqt, pht, L, nb):
    mesh = plsc.VectorSubcoreMesh(core_axis_name="c", subcore_axis_name="s",
                                  num_cores=NC, num_subcores=NS)
    k = pl.kernel(
        functools.partial(_gconv_body, L, nb),
        out_type=jax.ShapeDtypeStruct((xt_flat.shape[0],), jnp.float32),
        mesh=mesh,
        scratch_types=[
            pltpu.VMEM((W * D,), jnp.float32),
            pltpu.VMEM((BL * K,), jnp.float32),
            pltpu.VMEM((BL,), jnp.float32),
            pltpu.VMEM((BL,), jnp.float32),
            pltpu.VMEM((BL * D,), jnp.float32),
            pltpu.VMEM((16 * ILP,), jnp.int32),
        ],
        compiler_params=pltpu.CompilerParams(needs_layout_passes=False),
    )
    return k(xt_flat, kwt, fqt, pht)


def kernel(x, Ww, bw, Wk, bk, Wo):
    B, L, C = x.shape
    M = B * L
    BH = B * H
    nb = L // BL

    x2d = x.reshape(M, C)
    kern2d, freq2d, phase2d = _projections(
        x2d, Wk.T, bk[None, :], Ww.T, bw[None, :], BM=512)

    xt_flat = (x.reshape(B, L, H, D).transpose(0, 2, 1, 3)
               .reshape(BH * L * D))
    kwt = (kern2d.reshape(B, L, H, K).transpose(0, 2, 1, 3)
           .reshape(BH * L * K))
    fqt = freq2d.reshape(B, L, H).transpose(0, 2, 1).reshape(BH * L)
    pht = phase2d.reshape(B, L, H).transpose(0, 2, 1).reshape(BH * L)

    hid = _gather_conv(xt_flat, kwt, fqt, pht, L, nb)

    h2d = (hid.reshape(B, H, L, D).transpose(0, 2, 1, 3).reshape(M, C))
    out2d = _out_proj(h2d, Wo.T, BM=512)
    return out2d.reshape(B, L, C)


# speedup vs baseline: 1.5863x; 1.5863x over previous
import functools

import jax
import jax.numpy as jnp
from jax import lax
from jax.experimental import pallas as pl
from jax.experimental.pallas import tpu as pltpu
from jax.experimental.pallas import tpu_sc as plsc

H = 16
D = 64
K = 64
HALF_S = 16
S = 2 * HALF_S + 1
MAX_F = 16.0
MIN_F = 1.0
HALO = int(HALF_S * MAX_F + MAX_F)

NC = 2
NS = 16
NW = NC * NS

BL = 256
W = BL + 2 * HALO
ILP = 49

_RNE_MAGIC = 12582912.0


def _silu(v):
    return v * jax.nn.sigmoid(v)


def _proj_body(x_ref, wkT_ref, bk_ref, wwT_ref, bw_ref,
               kern_ref, freq_ref, phase_ref):
    xb = x_ref[...]
    kern_ref[...] = _silu(
        jnp.dot(xb, wkT_ref[...], preferred_element_type=jnp.float32)
        + bk_ref[...])
    wave = _silu(
        jnp.dot(xb, wwT_ref[...], preferred_element_type=jnp.float32)
        + bw_ref[...])
    freq_ref[...] = jax.nn.sigmoid(wave[:, :H]) * (MAX_F - MIN_F) + MIN_F
    phase_ref[...] = jnp.tanh(wave[:, H:]) * MAX_F


def _projections(x2d, WkT, bk, WwT, bw, BM):
    M, C = x2d.shape
    grid = (M // BM,)
    return pl.pallas_call(
        _proj_body,
        grid=grid,
        in_specs=[
            pl.BlockSpec((BM, C), lambda i: (i, 0)),
            pl.BlockSpec((C, H * K), lambda i: (0, 0)),
            pl.BlockSpec((1, H * K), lambda i: (0, 0)),
            pl.BlockSpec((C, 2 * H), lambda i: (0, 0)),
            pl.BlockSpec((1, 2 * H), lambda i: (0, 0)),
        ],
        out_specs=[
            pl.BlockSpec((BM, H * K), lambda i: (i, 0)),
            pl.BlockSpec((BM, H), lambda i: (i, 0)),
            pl.BlockSpec((BM, H), lambda i: (i, 0)),
        ],
        out_shape=[
            jax.ShapeDtypeStruct((M, H * K), jnp.float32),
            jax.ShapeDtypeStruct((M, H), jnp.float32),
            jax.ShapeDtypeStruct((M, H), jnp.float32),
        ],
    )(x2d, WkT, bk, WwT, bw)


def _out_body(h_ref, woT_ref, o_ref):
    o_ref[...] = _silu(
        jnp.dot(h_ref[...], woT_ref[...], preferred_element_type=jnp.float32))


def _out_proj(h2d, WoT, BM):
    M, C = h2d.shape
    return pl.pallas_call(
        _out_body,
        grid=(M // BM,),
        in_specs=[
            pl.BlockSpec((BM, C), lambda i: (i, 0)),
            pl.BlockSpec((C, C), lambda i: (0, 0)),
        ],
        out_specs=pl.BlockSpec((BM, C), lambda i: (i, 0)),
        out_shape=jax.ShapeDtypeStruct((M, C), jnp.float32),
    )(h2d, WoT)


def _gconv_body(L, nb, xt_ref, kw_ref, fq_ref, ph_ref, hid_ref,
                win, kwv, fqv, phv, outv, ilbuf):
    wid = lax.axis_index("s") * NC + lax.axis_index("c")
    iota = lax.iota(jnp.int32, 16)
    cols = [iota + 16 * j for j in range(4)]

    def block(blk, carry):
        l0 = blk * BL
        s0 = jnp.clip(l0 - HALO, 0, L - W)
        woff = pl.multiple_of(wid * (L * D) + s0 * D, 64)
        pltpu.sync_copy(xt_ref.at[pl.ds(woff, W * D)], win)
        koff = pl.multiple_of((wid * L + l0) * K, 256)
        pltpu.sync_copy(kw_ref.at[pl.ds(koff, BL * K)], kwv)
        boff = pl.multiple_of(wid * L + l0, 256)
        pltpu.sync_copy(fq_ref.at[pl.ds(boff, BL)], fqv)
        pltpu.sync_copy(ph_ref.at[pl.ds(boff, BL)], phv)

        def chunk(c, carry2):
            c16 = c * 16
            lvec = iota.astype(jnp.float32) + (l0 + c16).astype(jnp.float32)
            f = fqv[pl.ds(c16, 16)]
            p = phv[pl.ds(c16, 16)]
            base = lvec + p

            for s in range(S):
                pos = base + jnp.float32(s - HALF_S) * f
                r = (pos + _RNE_MAGIC) - _RNE_MAGIC
                r = jnp.clip(r, 0.0, jnp.float32(L - 1))
                il = r.astype(jnp.int32) - s0
                plsc.store_scatter(ilbuf, [iota * ILP + s], il * D)

            def gpass(s_lo, s_hi, first):
                @plsc.parallel_loop(0, 16, unroll=2)
                def per_l(l):
                    ilv = [ilbuf[pl.ds(l * ILP + 16 * t, 16)]
                           for t in range(3)]
                    wrow = [kwv[pl.ds((c16 + l) * K + 16 * t, 16)]
                            for t in range(3)]
                    accs = [jnp.zeros((16,), jnp.float32) for _ in range(8)]
                    for i, s in enumerate(range(s_lo, s_hi)):
                        ilb = jnp.full((16,), ilv[s // 16][s % 16], jnp.int32)
                        w = wrow[s // 16][s % 16]
                        for j in range(4):
                            v = plsc.load_gather(win, [ilb + cols[j]])
                            accs[(i % 2) * 4 + j] = (
                                accs[(i % 2) * 4 + j] + w * v)
                    for j in range(4):
                        acc = accs[j] + accs[4 + j]
                        if first:
                            outv[pl.ds((c16 + l) * D + 16 * j, 16)] = acc
                        else:
                            plsc.addupdate(
                                outv.at[pl.ds((c16 + l) * D + 16 * j, 16)],
                                acc)

            gpass(0, 17, True)
            gpass(17, S, False)
            return carry2

        lax.fori_loop(0, BL // 16, chunk, 0)
        ooff = pl.multiple_of((wid * L + l0) * D, 256)
        pltpu.sync_copy(outv, hid_ref.at[pl.ds(ooff, BL * D)])
        return carry

    lax.fori_loop(0, nb, block, 0)


def _gather_conv(xt_flat, kwt, fqt, pht, L, nb):
    mesh = plsc.VectorSubcoreMesh(core_axis_name="c", subcore_axis_name="s",
                                  num_cores=NC, num_subcores=NS)
    k = pl.kernel(
        functools.partial(_gconv_body, L, nb),
        out_type=jax.ShapeDtypeStruct((xt_flat.shape[0],), jnp.float32),
        mesh=mesh,
        scratch_types=[
            pltpu.VMEM((W * D,), jnp.float32),
            pltpu.VMEM((BL * K,), jnp.float32),
            pltpu.VMEM((BL,), jnp.float32),
            pltpu.VMEM((BL,), jnp.float32),
            pltpu.VMEM((BL * D,), jnp.float32),
            pltpu.VMEM((16 * ILP,), jnp.int32),
        ],
        compiler_params=pltpu.CompilerParams(needs_layout_passes=False),
    )
    return k(xt_flat, kwt, fqt, pht)


def kernel(x, Ww, bw, Wk, bk, Wo):
    B, L, C = x.shape
    M = B * L
    BH = B * H
    nb = L // BL

    x2d = x.reshape(M, C)
    kern2d, freq2d, phase2d = _projections(
        x2d, Wk.T, bk[None, :], Ww.T, bw[None, :], BM=512)

    xt_flat = (x.reshape(B, L, H, D).transpose(0, 2, 1, 3)
               .reshape(BH * L * D))
    kwt = (kern2d.reshape(B, L, H, K).transpose(0, 2, 1, 3)
           .reshape(BH * L * K))
    fqt = freq2d.reshape(B, L, H).transpose(0, 2, 1).reshape(BH * L)
    pht = phase2d.reshape(B, L, H).transpose(0, 2, 1).reshape(BH * L)

    hid = _gather_conv(xt_flat, kwt, fqt, pht, L, nb)

    h2d = (hid.reshape(B, H, L, D).transpose(0, 2, 1, 3).reshape(M, C))
    out2d = _out_proj(h2d, Wo.T, BM=512)
    return out2d.reshape(B, L, C)


# speedup vs baseline: 1.6356x; 1.0311x over previous
import functools

import jax
import jax.numpy as jnp
from jax import lax
from jax.experimental import pallas as pl
from jax.experimental.pallas import tpu as pltpu
from jax.experimental.pallas import tpu_sc as plsc

H = 16
D = 64
K = 64
HALF_S = 16
S = 2 * HALF_S + 1
MAX_F = 16.0
MIN_F = 1.0
HALO = int(HALF_S * MAX_F + MAX_F)

NC = 2
NS = 16
NW = NC * NS

BL = 512
W = BL + 2 * HALO + 8
DP = D // 2 + 1

_RNE_MAGIC = 12582912.0


def _silu(v):
    return v * jax.nn.sigmoid(v)


def _proj_body(x_ref, wkT_ref, bk_ref, wwT_ref, bw_ref,
               kern_ref, freq_ref, phase_ref):
    xb = x_ref[...]
    kern_ref[...] = _silu(
        jnp.dot(xb, wkT_ref[...], preferred_element_type=jnp.float32)
        + bk_ref[...])
    wave = _silu(
        jnp.dot(xb, wwT_ref[...], preferred_element_type=jnp.float32)
        + bw_ref[...])
    freq_ref[...] = jax.nn.sigmoid(wave[:, :H]) * (MAX_F - MIN_F) + MIN_F
    phase_ref[...] = jnp.tanh(wave[:, H:]) * MAX_F


def _projections(x2d, WkT, bk, WwT, bw, BM):
    M, C = x2d.shape
    grid = (M // BM,)
    return pl.pallas_call(
        _proj_body,
        grid=grid,
        in_specs=[
            pl.BlockSpec((BM, C), lambda i: (i, 0)),
            pl.BlockSpec((C, H * K), lambda i: (0, 0)),
            pl.BlockSpec((1, H * K), lambda i: (0, 0)),
            pl.BlockSpec((C, 2 * H), lambda i: (0, 0)),
            pl.BlockSpec((1, 2 * H), lambda i: (0, 0)),
        ],
        out_specs=[
            pl.BlockSpec((BM, H * K), lambda i: (i, 0)),
            pl.BlockSpec((BM, H), lambda i: (i, 0)),
            pl.BlockSpec((BM, H), lambda i: (i, 0)),
        ],
        out_shape=[
            jax.ShapeDtypeStruct((M, H * K), jnp.float32),
            jax.ShapeDtypeStruct((M, H), jnp.float32),
            jax.ShapeDtypeStruct((M, H), jnp.float32),
        ],
    )(x2d, WkT, bk, WwT, bw)


def _out_body(h_ref, woT_ref, o_ref):
    o_ref[...] = _silu(
        jnp.dot(h_ref[...], woT_ref[...], preferred_element_type=jnp.float32))


def _out_proj(h2d, WoT, BM):
    M, C = h2d.shape
    return pl.pallas_call(
        _out_body,
        grid=(M // BM,),
        in_specs=[
            pl.BlockSpec((BM, C), lambda i: (i, 0)),
            pl.BlockSpec((C, C), lambda i: (0, 0)),
        ],
        out_specs=pl.BlockSpec((BM, C), lambda i: (i, 0)),
        out_shape=jax.ShapeDtypeStruct((M, C), jnp.float32),
    )(h2d, WoT)


def _gconv_body(L, nb, xt_ref, fq_ref, ph_ref, kw_ref, hid_ref,
                win, fqv, phv, kwv, outv):
    wid = lax.axis_index("s") * NC + lax.axis_index("c")

    def block(blk, carry):
        l0 = blk * BL
        s0 = jnp.clip(l0 - HALO, 0, L - W) & ~7
        woff = pl.multiple_of(wid * (L * DP) + s0 * DP, 8)
        pltpu.sync_copy(xt_ref.at[pl.ds(woff, W * DP)], win)
        boff = pl.multiple_of((wid * nb + blk) * BL, 256)
        pltpu.sync_copy(fq_ref.at[pl.ds(boff, BL)], fqv)
        pltpu.sync_copy(ph_ref.at[pl.ds(boff, BL)], phv)
        koff = pl.multiple_of((wid * nb + blk) * (S * BL), 128)
        pltpu.sync_copy(kw_ref.at[pl.ds(koff, S * BL)], kwv)

        def chunk(c, carry2):
            c16 = c * 16
            lvec = lax.iota(jnp.int32, 16).astype(jnp.float32) + (
                (l0 + c16).astype(jnp.float32))
            f = fqv[pl.ds(c16, 16)]
            p = phv[pl.ds(c16, 16)]
            base = lvec + p

            def sample(s):
                pos = base + jnp.float32(s - HALF_S) * f
                r = (pos + _RNE_MAGIC) - _RNE_MAGIC
                r = jnp.clip(r, 0.0, jnp.float32(L - 1))
                il = r.astype(jnp.int32) - s0
                return il * DP, kwv[pl.ds(s * BL + c16, 16)]

            def tree_sum(vs):
                while len(vs) > 1:
                    nxt = [vs[i] + vs[i + 1] for i in range(0, len(vs) - 1, 2)]
                    if len(vs) % 2:
                        nxt.append(vs[-1])
                    vs = nxt
                return vs[0]

            def unpack(v):
                lo = plsc.bitcast(v << 16, jnp.float32)
                hi = plsc.bitcast(v & jnp.int32(-65536), jnp.float32)
                return lo, hi

            idxA = [sample(s) for s in range(17)]

            @plsc.parallel_loop(0, D // 2, unroll=2)
            def dlA(dp):
                los, his = [], []
                for fl, w in idxA:
                    lo, hi = unpack(plsc.load_gather(win, [fl + dp]))
                    los.append(w * lo)
                    his.append(w * hi)
                outv[pl.ds((2 * dp) * BL + c16, 16)] = tree_sum(los)
                outv[pl.ds((2 * dp + 1) * BL + c16, 16)] = tree_sum(his)

            idxB = [sample(s) for s in range(17, S)]

            @plsc.parallel_loop(0, D // 2, unroll=2)
            def dlB(dp):
                los, his = [], []
                for fl, w in idxB:
                    lo, hi = unpack(plsc.load_gather(win, [fl + dp]))
                    los.append(w * lo)
                    his.append(w * hi)
                plsc.addupdate(outv.at[pl.ds((2 * dp) * BL + c16, 16)],
                               tree_sum(los))
                plsc.addupdate(outv.at[pl.ds((2 * dp + 1) * BL + c16, 16)],
                               tree_sum(his))

            return carry2

        lax.fori_loop(0, BL // 16, chunk, 0)
        ooff = pl.multiple_of((wid * nb + blk) * (D * BL), 256)
        pltpu.sync_copy(outv, hid_ref.at[pl.ds(ooff, D * BL)])
        return carry

    lax.fori_loop(0, nb, block, 0)


def _gather_conv(xt_flat, fqt, pht, kwt, L, nb):
    BH = xt_flat.shape[0] // (L * DP)
    mesh = plsc.VectorSubcoreMesh(core_axis_name="c", subcore_axis_name="s",
                                  num_cores=NC, num_subcores=NS)
    k = pl.kernel(
        functools.partial(_gconv_body, L, nb),
        out_type=jax.ShapeDtypeStruct((BH * nb * D * BL,), jnp.float32),
        mesh=mesh,
        scratch_types=[
            pltpu.VMEM((W * DP,), jnp.int32),
            pltpu.VMEM((BL,), jnp.float32),
            pltpu.VMEM((BL,), jnp.float32),
            pltpu.VMEM((S * BL,), jnp.float32),
            pltpu.VMEM((D * BL,), jnp.float32),
        ],
        compiler_params=pltpu.CompilerParams(needs_layout_passes=False),
    )
    return k(xt_flat, fqt, pht, kwt)


def kernel(x, Ww, bw, Wk, bk, Wo):
    B, L, C = x.shape
    M = B * L
    BH = B * H
    nb = L // BL

    x2d = x.reshape(M, C)
    kern2d, freq2d, phase2d = _projections(
        x2d, Wk.T, bk[None, :], Ww.T, bw[None, :], BM=512)

    xt = x.reshape(B, L, H, D).transpose(0, 2, 1, 3)
    xp = lax.bitcast_convert_type(
        xt.astype(jnp.bfloat16).reshape(B, H, L, D // 2, 2), jnp.int32)
    xt_flat = jnp.pad(xp, ((0, 0), (0, 0), (0, 0), (0, 1))).reshape(
        BH * L * DP)
    fqt = (freq2d.reshape(B, L, H).transpose(0, 2, 1)
           .reshape(BH * nb * BL))
    pht = (phase2d.reshape(B, L, H).transpose(0, 2, 1)
           .reshape(BH * nb * BL))
    kwt = (kern2d.reshape(B, L, H, K)[:, :, :, :S]
           .reshape(B, nb, BL, H, S).transpose(0, 3, 1, 4, 2)
           .reshape(BH * nb * S * BL))

    hid = _gather_conv(xt_flat, fqt, pht, kwt, L, nb)

    h2d = (hid.reshape(B, H, nb, D, BL).transpose(0, 2, 4, 1, 3)
           .reshape(M, C))
    out2d = _out_proj(h2d, Wo.T, BM=512)
    return out2d.reshape(B, L, C)


# speedup vs baseline: 1.6630x; 1.0167x over previous
import functools

import jax
import jax.numpy as jnp
from jax import lax
from jax.experimental import pallas as pl
from jax.experimental.pallas import tpu as pltpu
from jax.experimental.pallas import tpu_sc as plsc

H = 16
D = 64
K = 64
HALF_S = 16
S = 2 * HALF_S + 1
MAX_F = 16.0
MIN_F = 1.0
HALO = int(HALF_S * MAX_F + MAX_F)

NC = 2
NS = 16
NW = NC * NS

BL = 512
W = BL + 2 * HALO + 8
DP = D // 2 + 1

_RNE_MAGIC = 12582912.0


def _silu(v):
    return v * jax.nn.sigmoid(v)


def _proj_body(x_ref, wkT_ref, bk_ref, wwT_ref, bw_ref,
               kern_ref, freq_ref, phase_ref):
    xb = x_ref[...]
    kern_ref[...] = _silu(
        jnp.dot(xb, wkT_ref[...], preferred_element_type=jnp.float32)
        + bk_ref[...])
    wave = _silu(
        jnp.dot(xb, wwT_ref[...], preferred_element_type=jnp.float32)
        + bw_ref[...])
    freq_ref[...] = jax.nn.sigmoid(wave[:, :H]) * (MAX_F - MIN_F) + MIN_F
    phase_ref[...] = jnp.tanh(wave[:, H:]) * MAX_F


def _projections(x2d, WkT, bk, WwT, bw, BM):
    M, C = x2d.shape
    grid = (M // BM,)
    return pl.pallas_call(
        _proj_body,
        grid=grid,
        in_specs=[
            pl.BlockSpec((BM, C), lambda i: (i, 0)),
            pl.BlockSpec((C, H * K), lambda i: (0, 0)),
            pl.BlockSpec((1, H * K), lambda i: (0, 0)),
            pl.BlockSpec((C, 2 * H), lambda i: (0, 0)),
            pl.BlockSpec((1, 2 * H), lambda i: (0, 0)),
        ],
        out_specs=[
            pl.BlockSpec((BM, H * K), lambda i: (i, 0)),
            pl.BlockSpec((BM, H), lambda i: (i, 0)),
            pl.BlockSpec((BM, H), lambda i: (i, 0)),
        ],
        out_shape=[
            jax.ShapeDtypeStruct((M, H * K), jnp.float32),
            jax.ShapeDtypeStruct((M, H), jnp.float32),
            jax.ShapeDtypeStruct((M, H), jnp.float32),
        ],
    )(x2d, WkT, bk, WwT, bw)


def _out_body(h_ref, woT_ref, o_ref):
    o_ref[...] = _silu(
        jnp.dot(h_ref[...], woT_ref[...], preferred_element_type=jnp.float32))


def _out_proj(h2d, WoT, BM):
    M, C = h2d.shape
    return pl.pallas_call(
        _out_body,
        grid=(M // BM,),
        in_specs=[
            pl.BlockSpec((BM, C), lambda i: (i, 0)),
            pl.BlockSpec((C, C), lambda i: (0, 0)),
        ],
        out_specs=pl.BlockSpec((BM, C), lambda i: (i, 0)),
        out_shape=jax.ShapeDtypeStruct((M, C), jnp.float32),
    )(h2d, WoT)


def _gconv_body(L, nb, l_start, xt_ref, fq_ref, ph_ref, kw_ref, hid_ref,
                win, fqv, phv, kwv, outv):
    wid = lax.axis_index("s") * NC + lax.axis_index("c")
    Lh = nb * BL

    def block(blk, carry):
        l0 = l_start + blk * BL
        s0 = jnp.clip(l0 - HALO, 0, L - W) & ~7
        woff = pl.multiple_of(wid * (L * DP) + s0 * DP, 8)
        pltpu.sync_copy(xt_ref.at[pl.ds(woff, W * DP)], win)
        boff = pl.multiple_of(wid * Lh + blk * BL, 256)
        pltpu.sync_copy(fq_ref.at[pl.ds(boff, BL)], fqv)
        pltpu.sync_copy(ph_ref.at[pl.ds(boff, BL)], phv)
        koff = pl.multiple_of((wid * nb + blk) * (S * BL), 128)
        pltpu.sync_copy(kw_ref.at[pl.ds(koff, S * BL)], kwv)

        def chunk(c, carry2):
            c16 = c * 16
            lvec = lax.iota(jnp.int32, 16).astype(jnp.float32) + (
                (l0 + c16).astype(jnp.float32))
            f = fqv[pl.ds(c16, 16)]
            p = phv[pl.ds(c16, 16)]
            base = lvec + p

            def sample(s):
                pos = base + jnp.float32(s - HALF_S) * f
                r = (pos + _RNE_MAGIC) - _RNE_MAGIC
                r = jnp.clip(r, 0.0, jnp.float32(L - 1))
                il = r.astype(jnp.int32) - s0
                return il * DP, kwv[pl.ds(s * BL + c16, 16)]

            def tree_sum(vs):
                while len(vs) > 1:
                    nxt = [vs[i] + vs[i + 1] for i in range(0, len(vs) - 1, 2)]
                    if len(vs) % 2:
                        nxt.append(vs[-1])
                    vs = nxt
                return vs[0]

            def unpack(v):
                lo = plsc.bitcast(v << 16, jnp.float32)
                hi = plsc.bitcast(v & jnp.int32(-65536), jnp.float32)
                return lo, hi

            idxA = [sample(s) for s in range(17)]

            @plsc.parallel_loop(0, D // 2, unroll=2)
            def dlA(dp):
                los, his = [], []
                for fl, w in idxA:
                    lo, hi = unpack(plsc.load_gather(win, [fl + dp]))
                    los.append(w * lo)
                    his.append(w * hi)
                outv[pl.ds((2 * dp) * BL + c16, 16)] = tree_sum(los)
                outv[pl.ds((2 * dp + 1) * BL + c16, 16)] = tree_sum(his)

            idxB = [sample(s) for s in range(17, S)]

            @plsc.parallel_loop(0, D // 2, unroll=2)
            def dlB(dp):
                los, his = [], []
                for fl, w in idxB:
                    lo, hi = unpack(plsc.load_gather(win, [fl + dp]))
                    los.append(w * lo)
                    his.append(w * hi)
                plsc.addupdate(outv.at[pl.ds((2 * dp) * BL + c16, 16)],
                               tree_sum(los))
                plsc.addupdate(outv.at[pl.ds((2 * dp + 1) * BL + c16, 16)],
                               tree_sum(his))

            return carry2

        lax.fori_loop(0, BL // 16, chunk, 0)
        ooff = pl.multiple_of((wid * nb + blk) * (D * BL), 256)
        pltpu.sync_copy(outv, hid_ref.at[pl.ds(ooff, D * BL)])
        return carry

    lax.fori_loop(0, nb, block, 0)


def _gather_conv(xt_flat, fqt, pht, kwt, L, nb, l_start):
    BH = xt_flat.shape[0] // (L * DP)
    mesh = plsc.VectorSubcoreMesh(core_axis_name="c", subcore_axis_name="s",
                                  num_cores=NC, num_subcores=NS)
    k = pl.kernel(
        functools.partial(_gconv_body, L, nb, l_start),
        out_type=jax.ShapeDtypeStruct((BH * nb * D * BL,), jnp.float32),
        name=f"gconv_l{l_start}",
        mesh=mesh,
        scratch_types=[
            pltpu.VMEM((W * DP,), jnp.int32),
            pltpu.VMEM((BL,), jnp.float32),
            pltpu.VMEM((BL,), jnp.float32),
            pltpu.VMEM((S * BL,), jnp.float32),
            pltpu.VMEM((D * BL,), jnp.float32),
        ],
        compiler_params=pltpu.CompilerParams(needs_layout_passes=False),
    )
    return k(xt_flat, fqt, pht, kwt)


def kernel(x, Ww, bw, Wk, bk, Wo):
    B, L, C = x.shape
    M = B * L
    BH = B * H
    Lh = L // 2
    nbh = Lh // BL

    x2d = x.reshape(M, C)
    xt = x.reshape(B, L, H, D).transpose(0, 2, 1, 3)
    xp = lax.bitcast_convert_type(
        xt.astype(jnp.bfloat16).reshape(B, H, L, D // 2, 2), jnp.int32)
    xt_flat = jnp.pad(xp, ((0, 0), (0, 0), (0, 0), (0, 1))).reshape(
        BH * L * DP)

    WkT, WwT, WoT = Wk.T, Ww.T, Wo.T
    bk2, bw2 = bk[None, :], bw[None, :]

    halves = []
    for half in range(2):
        xh = lax.dynamic_slice_in_dim(x, half * Lh, Lh, axis=1)
        kern2d, freq2d, phase2d = _projections(
            xh.reshape(B * Lh, C), WkT, bk2, WwT, bw2, BM=512)
        fqt = (freq2d.reshape(B, Lh, H).transpose(0, 2, 1)
               .reshape(BH * Lh))
        pht = (phase2d.reshape(B, Lh, H).transpose(0, 2, 1)
               .reshape(BH * Lh))
        kwt = (kern2d.reshape(B, Lh, H, K)[:, :, :, :S]
               .reshape(B, nbh, BL, H, S).transpose(0, 3, 1, 4, 2)
               .reshape(BH * nbh * S * BL))
        halves.append((fqt, pht, kwt))

    outs = []
    for half in range(2):
        fqt, pht, kwt = halves[half]
        hid = _gather_conv(xt_flat, fqt, pht, kwt, L, nbh, half * Lh)
        h2d = (hid.reshape(B, H, nbh, D, BL).transpose(0, 2, 4, 1, 3)
               .reshape(B * Lh, C))
        outs.append(_out_proj(h2d, WoT, BM=512).reshape(B, Lh, C))

    return jnp.concatenate(outs, axis=1)


# speedup vs baseline: 1.8612x; 1.1192x over previous
import functools

import jax
import jax.numpy as jnp
from jax import lax
from jax.experimental import pallas as pl
from jax.experimental.pallas import tpu as pltpu
from jax.experimental.pallas import tpu_sc as plsc

H = 16
D = 64
K = 64
HALF_S = 16
S = 2 * HALF_S + 1
MAX_F = 16.0
MIN_F = 1.0
HALO = int(HALF_S * MAX_F + MAX_F)

NC = 2
NS = 16
NW = NC * NS

BL = 512
W = BL + 2 * HALO + 8
DP = D // 2 + 1

_RNE_MAGIC = 12582912.0


def _silu(v):
    return v * jax.nn.sigmoid(v)


def _proj_body(x_ref, wkT_ref, bk_ref, wwT_ref, bw_ref,
               kwt_ref, freq_ref, phase_ref):
    xb = x_ref[...]
    kern = _silu(
        jnp.dot(xb, wkT_ref[...], preferred_element_type=jnp.float32)
        + bk_ref[...])
    kwt_ref[0, :, 0] = jnp.transpose(
        kern.reshape(kern.shape[0], H, K)[:, :, :S], (1, 2, 0))
    wave = _silu(
        jnp.dot(xb, wwT_ref[...], preferred_element_type=jnp.float32)
        + bw_ref[...])
    freq = jax.nn.sigmoid(wave[:, :H]) * (MAX_F - MIN_F) + MIN_F
    phase = jnp.tanh(wave[:, H:]) * MAX_F
    freq_ref[0] = jnp.transpose(freq, (1, 0))
    phase_ref[0] = jnp.transpose(phase, (1, 0))


def _projections(x2d, WkT, bk, WwT, bw, B, nbh):
    M, C = x2d.shape
    BM = M // (B * nbh)
    grid = (B * nbh,)

    def omap(i):
        return (i // nbh, 0, i % nbh, 0, 0)

    def omap3(i):
        return (i // nbh, 0, i % nbh)

    return pl.pallas_call(
        _proj_body,
        grid=grid,
        in_specs=[
            pl.BlockSpec((BM, C), lambda i: (i, 0)),
            pl.BlockSpec((C, H * K), lambda i: (0, 0)),
            pl.BlockSpec((1, H * K), lambda i: (0, 0)),
            pl.BlockSpec((C, 2 * H), lambda i: (0, 0)),
            pl.BlockSpec((1, 2 * H), lambda i: (0, 0)),
        ],
        out_specs=[
            pl.BlockSpec((1, H, 1, S, BM), omap),
            pl.BlockSpec((1, H, BM), omap3),
            pl.BlockSpec((1, H, BM), omap3),
        ],
        out_shape=[
            jax.ShapeDtypeStruct((B, H, nbh, S, BM), jnp.float32),
            jax.ShapeDtypeStruct((B, H, nbh * BM), jnp.float32),
            jax.ShapeDtypeStruct((B, H, nbh * BM), jnp.float32),
        ],
    )(x2d, WkT, bk, WwT, bw)


def _out_body(h_ref, woT_ref, o_ref):
    hb = jnp.transpose(h_ref[0, :, 0], (2, 0, 1))
    h2d = hb.reshape(hb.shape[0], H * D)
    o_ref[...] = _silu(
        jnp.dot(h2d, woT_ref[...], preferred_element_type=jnp.float32))


def _out_proj(hid5, WoT, B, nbh):
    _, _, _, _, BM = hid5.shape
    C = H * D
    return pl.pallas_call(
        _out_body,
        grid=(B * nbh,),
        in_specs=[
            pl.BlockSpec((1, H, 1, D, BM),
                         lambda i, n=nbh: (i // n, 0, i % n, 0, 0)),
            pl.BlockSpec((C, C), lambda i: (0, 0)),
        ],
        out_specs=pl.BlockSpec((BM, C), lambda i: (i, 0)),
        out_shape=jax.ShapeDtypeStruct((B * nbh * BM, C), jnp.float32),
    )(hid5, WoT)


def _gconv_body(L, nb, l_start, xt_ref, fq_ref, ph_ref, kw_ref, hid_ref,
                win, fqv, phv, kwv, outv):
    wid = lax.axis_index("s") * NC + lax.axis_index("c")
    Lh = nb * BL

    def block(blk, carry):
        l0 = l_start + blk * BL
        s0 = jnp.clip(l0 - HALO, 0, L - W) & ~7
        woff = pl.multiple_of(wid * (L * DP) + s0 * DP, 8)
        pltpu.sync_copy(xt_ref.at[pl.ds(woff, W * DP)], win)
        boff = pl.multiple_of(wid * Lh + blk * BL, 256)
        pltpu.sync_copy(fq_ref.at[pl.ds(boff, BL)], fqv)
        pltpu.sync_copy(ph_ref.at[pl.ds(boff, BL)], phv)
        koff = pl.multiple_of((wid * nb + blk) * (S * BL), 128)
        pltpu.sync_copy(kw_ref.at[pl.ds(koff, S * BL)], kwv)

        def chunk(c, carry2):
            c16 = c * 16
            lvec = lax.iota(jnp.int32, 16).astype(jnp.float32) + (
                (l0 + c16).astype(jnp.float32))
            f = fqv[pl.ds(c16, 16)]
            p = phv[pl.ds(c16, 16)]
            base = lvec + p

            def sample(s):
                pos = base + jnp.float32(s - HALF_S) * f
                r = (pos + _RNE_MAGIC) - _RNE_MAGIC
                r = jnp.clip(r, 0.0, jnp.float32(L - 1))
                il = r.astype(jnp.int32) - s0
                return il * DP, kwv[pl.ds(s * BL + c16, 16)]

            def tree_sum(vs):
                while len(vs) > 1:
                    nxt = [vs[i] + vs[i + 1] for i in range(0, len(vs) - 1, 2)]
                    if len(vs) % 2:
                        nxt.append(vs[-1])
                    vs = nxt
                return vs[0]

            def unpack(v):
                lo = plsc.bitcast(v << 16, jnp.float32)
                hi = plsc.bitcast(v & jnp.int32(-65536), jnp.float32)
                return lo, hi

            idxA = [sample(s) for s in range(17)]

            @plsc.parallel_loop(0, D // 2, unroll=2)
            def dlA(dp):
                los, his = [], []
                for fl, w in idxA:
                    lo, hi = unpack(plsc.load_gather(win, [fl + dp]))
                    los.append(w * lo)
                    his.append(w * hi)
                outv[pl.ds((2 * dp) * BL + c16, 16)] = tree_sum(los)
                outv[pl.ds((2 * dp + 1) * BL + c16, 16)] = tree_sum(his)

            idxB = [sample(s) for s in range(17, S)]

            @plsc.parallel_loop(0, D // 2, unroll=2)
            def dlB(dp):
                los, his = [], []
                for fl, w in idxB:
                    lo, hi = unpack(plsc.load_gather(win, [fl + dp]))
                    los.append(w * lo)
                    his.append(w * hi)
                plsc.addupdate(outv.at[pl.ds((2 * dp) * BL + c16, 16)],
                               tree_sum(los))
                plsc.addupdate(outv.at[pl.ds((2 * dp + 1) * BL + c16, 16)],
                               tree_sum(his))

            return carry2

        lax.fori_loop(0, BL // 16, chunk, 0)
        ooff = pl.multiple_of((wid * nb + blk) * (D * BL), 256)
        pltpu.sync_copy(outv, hid_ref.at[pl.ds(ooff, D * BL)])
        return carry

    lax.fori_loop(0, nb, block, 0)


def _gather_conv(xt_flat, fqt, pht, kwt, L, nb, l_start):
    BH = xt_flat.shape[0] // (L * DP)
    mesh = plsc.VectorSubcoreMesh(core_axis_name="c", subcore_axis_name="s",
                                  num_cores=NC, num_subcores=NS)
    k = pl.kernel(
        functools.partial(_gconv_body, L, nb, l_start),
        out_type=jax.ShapeDtypeStruct((BH * nb * D * BL,), jnp.float32),
        name=f"gconv_l{l_start}",
        mesh=mesh,
        scratch_types=[
            pltpu.VMEM((W * DP,), jnp.int32),
            pltpu.VMEM((BL,), jnp.float32),
            pltpu.VMEM((BL,), jnp.float32),
            pltpu.VMEM((S * BL,), jnp.float32),
            pltpu.VMEM((D * BL,), jnp.float32),
        ],
        compiler_params=pltpu.CompilerParams(needs_layout_passes=False),
    )
    return k(xt_flat, fqt, pht, kwt)


def kernel(x, Ww, bw, Wk, bk, Wo):
    B, L, C = x.shape
    BH = B * H
    Lh = L // 2
    nbh = Lh // BL

    xt = x.reshape(B, L, H, D).transpose(0, 2, 1, 3)
    xp = lax.bitcast_convert_type(
        xt.astype(jnp.bfloat16).reshape(B, H, L, D // 2, 2), jnp.int32)
    xt_flat = jnp.pad(xp, ((0, 0), (0, 0), (0, 0), (0, 1))).reshape(
        BH * L * DP)

    WkT, WwT, WoT = Wk.T, Ww.T, Wo.T
    bk2, bw2 = bk[None, :], bw[None, :]

    halves = []
    for half in range(2):
        xh = lax.dynamic_slice_in_dim(x, half * Lh, Lh, axis=1)
        halves.append(_projections(
            xh.reshape(B * Lh, C), WkT, bk2, WwT, bw2, B, nbh))

    outs = []
    for half in range(2):
        kwt5, fqt4, pht4 = halves[half]
        hid = _gather_conv(xt_flat, fqt4.reshape(BH * Lh),
                           pht4.reshape(BH * Lh),
                           kwt5.reshape(BH * nbh * S * BL), L, nbh,
                           half * Lh)
        outs.append(
            _out_proj(hid.reshape(B, H, nbh, D, BL), WoT, B, nbh)
            .reshape(B, Lh, C))

    return jnp.concatenate(outs, axis=1)
